# baseline (speedup 1.0000x reference)
import jax
import jax.numpy as jnp
from jax import lax
from jax.experimental import pallas as pl
from jax.experimental.pallas import tpu as pltpu
from jax.experimental.pallas import tpu_sc as plsc

N = 10000
E = 320000
D = 128
H = 4
C = 128

NSC = 2
NTEC = 16
LANES = 16

NN = 10112
NH = NN * H
EP = 331776
EPT = EP // NTEC
A_CH = 768
A_NCH = EPT // A_CH
B_CH = 32
SB = 768
IC = SB // B_CH
SCN = EPT // SB
STRIDE = NH // NTEC
ROWS_T = NN // NTEC
MM_BLK = 1264
W_COLS = 1152


def _mm_body(x_ref, w_ref, o_ref):
    o_ref[...] = jnp.dot(x_ref[...], w_ref[...],
                         preferred_element_type=jnp.float32)


def _dense_matmul(x_pad, w_cat):
    return pl.pallas_call(
        _mm_body,
        grid=(NN // MM_BLK,),
        in_specs=[
            pl.BlockSpec((MM_BLK, D), lambda i: (i, 0)),
            pl.BlockSpec((D, W_COLS), lambda i: (0, 0)),
        ],
        out_specs=pl.BlockSpec((MM_BLK, W_COLS), lambda i: (i, 0)),
        out_shape=jax.ShapeDtypeStruct((NN, W_COLS), jnp.float32),
    )(x_pad, w_cat)


def _lrelu_exp(a):
    return jnp.exp(jnp.where(a >= 0.0, a, 0.2 * a))


def _edge_logits(asrc_t, adst_t, srcb, dstb, j, h):
    sv = srcb[pl.ds(j * LANES, LANES)] * H + h
    dv = dstb[pl.ds(j * LANES, LANES)] * H + h
    a = plsc.load_gather(asrc_t, [sv]) + plsc.load_gather(adst_t, [dv])
    return _lrelu_exp(a), dv


def _sc_a(asrc_flat, adst_flat, srcp, dstp):
    mesh = plsc.VectorSubcoreMesh(core_axis_name="c", subcore_axis_name="s")

    def body(asrc_hbm, adst_hbm, src_hbm, dst_hbm,
             aw_hbm, spart, denom_hbm,
             asrc_t, adst_t, dpart, srcb, dstb, ab, redbuf):
        cid = lax.axis_index("c")
        tid = lax.axis_index("s")
        zero16 = jnp.zeros((LANES,), jnp.float32)

        pltpu.sync_copy(asrc_hbm.at[pl.ds(cid * NH, NH)], asrc_t)
        pltpu.sync_copy(adst_hbm.at[pl.ds(cid * NH, NH)], adst_t)

        def _zero(i, _):
            dpart[pl.ds(i * LANES, LANES)] = zero16
            return 0
        lax.fori_loop(0, NH // LANES, _zero, 0)

        def _p1(ch, _):
            base = tid * EPT + ch * A_CH
            pltpu.sync_copy(src_hbm.at[pl.ds(base, A_CH)], srcb)
            pltpu.sync_copy(dst_hbm.at[pl.ds(base, A_CH)], dstb)

            def _vreg(j, _):
                for h in range(H):
                    e, dv = _edge_logits(asrc_t, adst_t, srcb, dstb, j, h)
                    plsc.addupdate_scatter(dpart, [dv], e)
                return 0
            lax.fori_loop(0, A_CH // LANES, _vreg, 0)
            return 0
        lax.fori_loop(0, A_NCH, _p1, 0)

        cbase = cid * NTEC * NH
        pltpu.sync_copy(dpart, spart.at[pl.ds(cbase + tid * NH, NH)])
        plsc.subcore_barrier()

        def _zr(i, _):
            redbuf[pl.ds(i * LANES, LANES)] = zero16
            return 0
        lax.fori_loop(0, STRIDE // LANES, _zr, 0)
        for p in range(NTEC):
            pltpu.sync_copy(
                spart.at[pl.ds(cbase + p * NH + tid * STRIDE, STRIDE)],
                dpart.at[pl.ds(0, STRIDE)])

            def _acc(i, _):
                redbuf[pl.ds(i * LANES, LANES)] = (
                    redbuf[pl.ds(i * LANES, LANES)]
                    + dpart[pl.ds(i * LANES, LANES)])
                return 0
            lax.fori_loop(0, STRIDE // LANES, _acc, 0)
        pltpu.sync_copy(redbuf,
                        denom_hbm.at[pl.ds(cid * NH + tid * STRIDE, STRIDE)])
        plsc.subcore_barrier()

        pltpu.sync_copy(denom_hbm.at[pl.ds(cid * NH, NH)], dpart)

        iota = lax.broadcasted_iota(jnp.int32, (LANES,), 0)

        def _p2(ch, _):
            base = tid * EPT + ch * A_CH
            pltpu.sync_copy(src_hbm.at[pl.ds(base, A_CH)], srcb)
            pltpu.sync_copy(dst_hbm.at[pl.ds(base, A_CH)], dstb)

            def _vreg(j, _):
                for h in range(H):
                    e, dv = _edge_logits(asrc_t, adst_t, srcb, dstb, j, h)
                    d = plsc.load_gather(dpart, [dv])
                    aw = e / (d + 1e-16) * (1.0 / H)
                    pos = (j * LANES + iota) * H + h
                    plsc.store_scatter(ab, [pos], aw)
                return 0
            lax.fori_loop(0, A_CH // LANES, _vreg, 0)
            pltpu.sync_copy(ab, aw_hbm.at[pl.ds(cid * EP * H + base * H,
                                                A_CH * H)])
            return 0
        lax.fori_loop(0, A_NCH, _p2, 0)

    return pl.kernel(
        body,
        out_type=(
            jax.ShapeDtypeStruct((NSC * EP * H,), jnp.float32),
            jax.ShapeDtypeStruct((NSC * NTEC * NH,), jnp.float32),
            jax.ShapeDtypeStruct((NSC * NH,), jnp.float32),
        ),
        mesh=mesh,
        compiler_params=pltpu.CompilerParams(needs_layout_passes=False),
        scratch_types=[
            pltpu.VMEM((NH,), jnp.float32),
            pltpu.VMEM((NH,), jnp.float32),
            pltpu.VMEM((NH,), jnp.float32),
            pltpu.VMEM((A_CH,), jnp.int32),
            pltpu.VMEM((A_CH,), jnp.int32),
            pltpu.VMEM((A_CH * H,), jnp.float32),
            pltpu.VMEM((STRIDE,), jnp.float32),
        ],
    )(asrc_flat, adst_flat, srcp, dstp)


def _sc_b(h2, aw_flat, srcp2, dstp2d):
    mesh = plsc.VectorSubcoreMesh(core_axis_name="c", subcore_axis_name="s")

    def body(h_hbm, aw_hbm, src_hbm, dst_hbm, out_hbm,
             sbufS, dbufS, awb0, awb1,
             hr0, hr1, ms0, ms1, out_sh,
             gsem0, gsem1, ssem0, ssem1, asem0, asem1):
        cid = lax.axis_index("c")
        tid = lax.axis_index("s")
        zero16 = jnp.zeros((LANES,), jnp.float32)
        hr = (hr0, hr1)
        ms = (ms0, ms1)
        awb = (awb0, awb1)
        gsem = (gsem0, gsem1)
        ssem = (ssem0, ssem1)
        asem = (asem0, asem1)

        def _zm(i, _):
            for k in range(C // LANES):
                ms0[i, pl.ds(k * LANES, LANES)] = zero16
                ms1[i, pl.ds(k * LANES, LANES)] = zero16
            return 0
        lax.fori_loop(0, B_CH, _zm, 0)
        rbase = tid * ROWS_T
        nfull = ROWS_T // B_CH
        for q in range(nfull):
            pltpu.sync_copy(ms0, out_sh.at[pl.ds(rbase + q * B_CH, B_CH)])
        rem = ROWS_T - nfull * B_CH
        if rem:
            pltpu.sync_copy(ms0.at[pl.ds(0, rem)],
                            out_sh.at[pl.ds(rbase + nfull * B_CH, rem)])
        plsc.subcore_barrier()

        def _issue_gather(i, par):
            pltpu.async_copy(
                h_hbm.at[sbufS.at[pl.ds(i * B_CH, B_CH)]], hr[par],
                gsem[par])

        def _wait_gather(par):
            pltpu.make_async_copy(
                h_hbm.at[pl.ds(0, B_CH)], hr[par], gsem[par]).wait()

        def _issue_aw(awoff, i, par):
            pltpu.async_copy(
                aw_hbm.at[pl.ds(awoff + i * (B_CH * H), B_CH * H)],
                awb[par], asem[par])

        def _wait_aw(par):
            pltpu.make_async_copy(
                aw_hbm.at[pl.ds(0, B_CH * H)], awb[par], asem[par]).wait()

        def _wait_scatter(par):
            pltpu.make_async_copy(
                ms[par], out_sh.at[pl.ds(0, B_CH)], ssem[par]).wait()

        def _compute(i, par):
            _wait_aw(par)

            himask = jnp.full((LANES,), -65536, jnp.int32)

            def _edge(q, _):
                b0 = q * 2
                base = jnp.full((LANES,), b0 * H, jnp.int32)
                ws = [plsc.load_gather(awb[par], [base + e * H + h])
                      for e in range(2) for h in range(H)]
                for e in range(2):
                    acc = [zero16] * (C // LANES)
                    for h in range(H):
                        w = ws[e * H + h]
                        for j in range(C // (2 * LANES)):
                            v = plsc.bitcast(
                                hr[par][b0 + e,
                                        pl.ds(h * (C // 2) + j * LANES,
                                              LANES)],
                                jnp.int32)
                            u0 = plsc.bitcast(v << 16, jnp.float32)
                            u1 = plsc.bitcast(v & himask, jnp.float32)
                            acc[2 * j] = acc[2 * j] + w * u0
                            acc[2 * j + 1] = acc[2 * j + 1] + w * u1
                    for k in range(C // LANES):
                        ms[par][b0 + e, pl.ds(k * LANES, LANES)] = acc[k]
                return 0
            lax.fori_loop(0, B_CH // 2, _edge, 0)
            pltpu.async_copy(ms[par], out_sh.at[dbufS.at[i]], ssem[par],
                             add=True)

        def _super(s, _):
            sgbase = tid * EPT + s * SB
            awoff = cid * EP * H + sgbase * H
            drow = pl.multiple_of(tid * (EPT // B_CH) + s * IC, 8)
            pltpu.sync_copy(src_hbm.at[pl.ds(cid * EP + sgbase, SB)], sbufS)

            @pl.when(s > 0)
            def _():
                _wait_scatter(0)
                _wait_scatter(1)
            pltpu.sync_copy(dst_hbm.at[pl.ds(drow, IC), :], dbufS)
            _issue_gather(0, 0)
            _issue_gather(1, 1)
            _issue_aw(awoff, 0, 0)
            _issue_aw(awoff, 1, 1)

            def _chunk(i, par):
                _wait_gather(par)

                @pl.when(i >= 2)
                def _():
                    _wait_scatter(par)
                _compute(i, par)

            def _pair(p, _):
                for par in range(2):
                    i = p * 2 + par
                    _chunk(i, par)
                    _issue_gather(i + 2, par)
                    _issue_aw(awoff, i + 2, par)
                return 0

            lax.fori_loop(0, (IC - 2) // 2, _pair, 0)
            for i in (IC - 2, IC - 1):
                _chunk(i, i % 2)
            return 0
        lax.fori_loop(0, SCN, _super, 0)
        _wait_scatter(0)
        _wait_scatter(1)
        plsc.subcore_barrier()

        pltpu.sync_copy(out_sh.at[pl.ds(rbase, ROWS_T)],
                        out_hbm.at[pl.ds(cid * NN + rbase, ROWS_T)])

    return pl.kernel(
        body,
        out_type=jax.ShapeDtypeStruct((NSC * NN, C), jnp.float32),
        mesh=mesh,
        compiler_params=pltpu.CompilerParams(needs_layout_passes=False),
        scratch_types=[
            pltpu.VMEM((SB,), jnp.int32),
            pltpu.VMEM((IC, B_CH), jnp.int32),
            pltpu.VMEM((B_CH * H,), jnp.float32),
            pltpu.VMEM((B_CH * H,), jnp.float32),
            pltpu.VMEM((B_CH, H * C // 2), jnp.float32),
            pltpu.VMEM((B_CH, H * C // 2), jnp.float32),
            pltpu.VMEM((B_CH, C), jnp.float32),
            pltpu.VMEM((B_CH, C), jnp.float32),
            pltpu.VMEM_SHARED((NN, C), jnp.float32),
            pltpu.SemaphoreType.DMA,
            pltpu.SemaphoreType.DMA,
            pltpu.SemaphoreType.DMA,
            pltpu.SemaphoreType.DMA,
            pltpu.SemaphoreType.DMA,
            pltpu.SemaphoreType.DMA,
        ],
    )(h2, aw_flat, srcp2, dstp2d)


def kernel(x, edge_index, W_m, a_src_m, a_dst_m, b_m,
           W_s, a_src_s, a_dst_s, b_s):
    f32 = jnp.float32

    def fold(W, a):
        return (W.reshape(D, H, C) * a[None, :, :]).sum(-1)

    w_cat = jnp.concatenate(
        [W_m, W_s,
         fold(W_m, a_src_m), fold(W_m, a_dst_m),
         fold(W_s, a_src_s), fold(W_s, a_dst_s),
         jnp.zeros((D, W_COLS - 2 * H * C - 4 * H), f32)], axis=1)

    x_pad = jnp.zeros((NN, D), f32).at[:N].set(x)
    h_all = _dense_matmul(x_pad, w_cat)

    h2 = jnp.concatenate([h_all[:, :H * C], h_all[:, H * C:2 * H * C]], 0)
    base = 2 * H * C
    asrc_flat = jnp.concatenate(
        [h_all[:, base:base + H].reshape(-1),
         h_all[:, base + 2 * H:base + 3 * H].reshape(-1)])
    adst_flat = jnp.concatenate(
        [h_all[:, base + H:base + 2 * H].reshape(-1),
         h_all[:, base + 3 * H:base + 4 * H].reshape(-1)])

    loop = jnp.arange(N, dtype=jnp.int32)
    padv = jnp.full((EP - E - N,), N, jnp.int32)
    srcp = jnp.concatenate([edge_index[0], loop, padv])
    dstp = jnp.concatenate([edge_index[1], loop, padv])
    srcp2 = jnp.concatenate([srcp, srcp + NN])
    dstp2d = dstp.reshape(EP // B_CH, B_CH)

    aw_flat, _, _ = _sc_a(asrc_flat, adst_flat, srcp, dstp)
    h2bf = h2.astype(jnp.bfloat16).reshape(NSC * NN, H * C // 2, 2)
    h2w = jax.lax.bitcast_convert_type(h2bf, jnp.float32)
    out2 = _sc_b(h2w, aw_flat, srcp2, dstp2d)

    blk = jnp.arange(C, dtype=jnp.int32)
    j2, r = blk // 32, blk % 32
    srccol = j2 * 32 + jnp.where(r % 2 == 0, r // 2, 16 + r // 2)
    out2 = out2[:, srccol]

    z_mean = out2[:N] + b_m[None, :]
    z_logstd = out2[NN:NN + N] + b_s[None, :]
    return (z_mean, z_logstd)


# speedup vs baseline: 1.0410x; 1.0410x over previous
import jax
import jax.numpy as jnp
from jax import lax
from jax.experimental import pallas as pl
from jax.experimental.pallas import tpu as pltpu
from jax.experimental.pallas import tpu_sc as plsc

N = 10000
E = 320000
D = 128
H = 4
C = 128

NSC = 2
NTEC = 16
LANES = 16

NN = 10112
NH = NN * H
EP = 331776
EPT = EP // NTEC
A_CH = 768
A_NCH = EPT // A_CH
B_CH = 32
SB = 768
IC = SB // B_CH
SCN = EPT // SB
STRIDE = NH // NTEC
ROWS_T = NN // NTEC
MM_BLK = 1264
W_COLS = 1152


def _mm_body(x_ref, w_ref, hbf_ref, al_ref):
    res = jnp.dot(x_ref[...], w_ref[...], preferred_element_type=jnp.float32)
    hbf_ref[0] = res[:, :H * C].astype(jnp.bfloat16)
    hbf_ref[1] = res[:, H * C:2 * H * C].astype(jnp.bfloat16)
    al_ref[...] = res[:, 2 * H * C:]


def _dense_matmul(x_pad, w_cat):
    return pl.pallas_call(
        _mm_body,
        grid=(NN // MM_BLK,),
        in_specs=[
            pl.BlockSpec((MM_BLK, D), lambda i: (i, 0)),
            pl.BlockSpec((D, W_COLS), lambda i: (0, 0)),
        ],
        out_specs=[
            pl.BlockSpec((NSC, MM_BLK, H * C), lambda i: (0, i, 0)),
            pl.BlockSpec((MM_BLK, W_COLS - 2 * H * C), lambda i: (i, 0)),
        ],
        out_shape=[
            jax.ShapeDtypeStruct((NSC, NN, H * C), jnp.bfloat16),
            jax.ShapeDtypeStruct((NN, W_COLS - 2 * H * C), jnp.float32),
        ],
    )(x_pad, w_cat)


def _lrelu_exp(a):
    return jnp.exp(jnp.where(a >= 0.0, a, 0.2 * a))


def _edge_logits(asrc_t, adst_t, srcb, dstb, j, h):
    sv = srcb[pl.ds(j * LANES, LANES)] * H + h
    dv = dstb[pl.ds(j * LANES, LANES)] * H + h
    a = plsc.load_gather(asrc_t, [sv]) + plsc.load_gather(adst_t, [dv])
    return _lrelu_exp(a), dv


def _sc_a(asrc_flat, adst_flat, srcp, dstp):
    mesh = plsc.VectorSubcoreMesh(core_axis_name="c", subcore_axis_name="s")

    def body(asrc_hbm, adst_hbm, src_hbm, dst_hbm,
             aw_hbm, spart, denom_hbm,
             asrc_t, adst_t, dpart, srcb, dstb, ab, redbuf):
        cid = lax.axis_index("c")
        tid = lax.axis_index("s")
        zero16 = jnp.zeros((LANES,), jnp.float32)

        pltpu.sync_copy(asrc_hbm.at[pl.ds(cid * NH, NH)], asrc_t)
        pltpu.sync_copy(adst_hbm.at[pl.ds(cid * NH, NH)], adst_t)

        def _zero(i, _):
            dpart[pl.ds(i * LANES, LANES)] = zero16
            return 0
        lax.fori_loop(0, NH // LANES, _zero, 0)

        def _p1(ch, _):
            base = tid * EPT + ch * A_CH
            pltpu.sync_copy(src_hbm.at[pl.ds(base, A_CH)], srcb)
            pltpu.sync_copy(dst_hbm.at[pl.ds(base, A_CH)], dstb)

            def _vreg(j, _):
                for h in range(H):
                    e, dv = _edge_logits(asrc_t, adst_t, srcb, dstb, j, h)
                    plsc.addupdate_scatter(dpart, [dv], e)
                return 0
            lax.fori_loop(0, A_CH // LANES, _vreg, 0)
            return 0
        lax.fori_loop(0, A_NCH, _p1, 0)

        cbase = cid * NTEC * NH
        pltpu.sync_copy(dpart, spart.at[pl.ds(cbase + tid * NH, NH)])
        plsc.subcore_barrier()

        def _zr(i, _):
            redbuf[pl.ds(i * LANES, LANES)] = zero16
            return 0
        lax.fori_loop(0, STRIDE // LANES, _zr, 0)
        for p in range(NTEC):
            pltpu.sync_copy(
                spart.at[pl.ds(cbase + p * NH + tid * STRIDE, STRIDE)],
                dpart.at[pl.ds(0, STRIDE)])

            def _acc(i, _):
                redbuf[pl.ds(i * LANES, LANES)] = (
                    redbuf[pl.ds(i * LANES, LANES)]
                    + dpart[pl.ds(i * LANES, LANES)])
                return 0
            lax.fori_loop(0, STRIDE // LANES, _acc, 0)
        pltpu.sync_copy(redbuf,
                        denom_hbm.at[pl.ds(cid * NH + tid * STRIDE, STRIDE)])
        plsc.subcore_barrier()

        pltpu.sync_copy(denom_hbm.at[pl.ds(cid * NH, NH)], dpart)

        iota = lax.broadcasted_iota(jnp.int32, (LANES,), 0)

        def _p2(ch, _):
            base = tid * EPT + ch * A_CH
            pltpu.sync_copy(src_hbm.at[pl.ds(base, A_CH)], srcb)
            pltpu.sync_copy(dst_hbm.at[pl.ds(base, A_CH)], dstb)

            def _vreg(j, _):
                for h in range(H):
                    e, dv = _edge_logits(asrc_t, adst_t, srcb, dstb, j, h)
                    d = plsc.load_gather(dpart, [dv])
                    aw = e / (d + 1e-16) * (1.0 / H)
                    pos = (j * LANES + iota) * H + h
                    plsc.store_scatter(ab, [pos], aw)
                return 0
            lax.fori_loop(0, A_CH // LANES, _vreg, 0)
            pltpu.sync_copy(ab, aw_hbm.at[pl.ds(cid * EP * H + base * H,
                                                A_CH * H)])
            return 0
        lax.fori_loop(0, A_NCH, _p2, 0)

    return pl.kernel(
        body,
        out_type=(
            jax.ShapeDtypeStruct((NSC * EP * H,), jnp.float32),
            jax.ShapeDtypeStruct((NSC * NTEC * NH,), jnp.float32),
            jax.ShapeDtypeStruct((NSC * NH,), jnp.float32),
        ),
        mesh=mesh,
        compiler_params=pltpu.CompilerParams(needs_layout_passes=False),
        scratch_types=[
            pltpu.VMEM((NH,), jnp.float32),
            pltpu.VMEM((NH,), jnp.float32),
            pltpu.VMEM((NH,), jnp.float32),
            pltpu.VMEM((A_CH,), jnp.int32),
            pltpu.VMEM((A_CH,), jnp.int32),
            pltpu.VMEM((A_CH * H,), jnp.float32),
            pltpu.VMEM((STRIDE,), jnp.float32),
        ],
    )(asrc_flat, adst_flat, srcp, dstp)


def _sc_b(h2, aw_flat, srcp2, dstp2d):
    mesh = plsc.VectorSubcoreMesh(core_axis_name="c", subcore_axis_name="s")

    def body(h_hbm, aw_hbm, src_hbm, dst_hbm, out_hbm,
             sbufS, dbufS, awb0, awb1,
             hr0, hr1, ms0, ms1, out_sh,
             gsem0, gsem1, ssem0, ssem1, asem0, asem1):
        cid = lax.axis_index("c")
        tid = lax.axis_index("s")
        zero16 = jnp.zeros((LANES,), jnp.float32)
        hr = (hr0, hr1)
        ms = (ms0, ms1)
        awb = (awb0, awb1)
        gsem = (gsem0, gsem1)
        ssem = (ssem0, ssem1)
        asem = (asem0, asem1)

        def _zm(i, _):
            for k in range(C // LANES):
                ms0[i, pl.ds(k * LANES, LANES)] = zero16
                ms1[i, pl.ds(k * LANES, LANES)] = zero16
            return 0
        lax.fori_loop(0, B_CH, _zm, 0)
        rbase = tid * ROWS_T
        nfull = ROWS_T // B_CH
        for q in range(nfull):
            pltpu.sync_copy(ms0, out_sh.at[pl.ds(rbase + q * B_CH, B_CH)])
        rem = ROWS_T - nfull * B_CH
        if rem:
            pltpu.sync_copy(ms0.at[pl.ds(0, rem)],
                            out_sh.at[pl.ds(rbase + nfull * B_CH, rem)])
        plsc.subcore_barrier()

        def _issue_gather(i, par):
            pltpu.async_copy(
                h_hbm.at[sbufS.at[pl.ds(i * B_CH, B_CH)]], hr[par],
                gsem[par])

        def _wait_gather(par):
            pltpu.make_async_copy(
                h_hbm.at[pl.ds(0, B_CH)], hr[par], gsem[par]).wait()

        def _issue_aw(awoff, i, par):
            pltpu.async_copy(
                aw_hbm.at[pl.ds(awoff + i * (B_CH * H), B_CH * H)],
                awb[par], asem[par])

        def _wait_aw(par):
            pltpu.make_async_copy(
                aw_hbm.at[pl.ds(0, B_CH * H)], awb[par], asem[par]).wait()

        def _wait_scatter(par):
            pltpu.make_async_copy(
                ms[par], out_sh.at[pl.ds(0, B_CH)], ssem[par]).wait()

        def _compute(i, par):
            _wait_aw(par)

            himask = jnp.full((LANES,), -65536, jnp.int32)

            def _edge(q, _):
                b0 = q * 2
                base = jnp.full((LANES,), b0 * H, jnp.int32)
                ws = [plsc.load_gather(awb[par], [base + e * H + h])
                      for e in range(2) for h in range(H)]
                for e in range(2):
                    acc = [zero16] * (C // LANES)
                    for h in range(H):
                        w = ws[e * H + h]
                        for j in range(C // (2 * LANES)):
                            v = plsc.bitcast(
                                hr[par][b0 + e,
                                        pl.ds(h * (C // 2) + j * LANES,
                                              LANES)],
                                jnp.int32)
                            u0 = plsc.bitcast(v << 16, jnp.float32)
                            u1 = plsc.bitcast(v & himask, jnp.float32)
                            acc[2 * j] = acc[2 * j] + w * u0
                            acc[2 * j + 1] = acc[2 * j + 1] + w * u1
                    for k in range(C // LANES):
                        ms[par][b0 + e, pl.ds(k * LANES, LANES)] = acc[k]
                return 0
            lax.fori_loop(0, B_CH // 2, _edge, 0)
            pltpu.async_copy(ms[par], out_sh.at[dbufS.at[i]], ssem[par],
                             add=True)

        def _super(s, _):
            sgbase = tid * EPT + s * SB
            awoff = cid * EP * H + sgbase * H
            drow = pl.multiple_of(tid * (EPT // B_CH) + s * IC, 8)
            pltpu.sync_copy(src_hbm.at[pl.ds(cid * EP + sgbase, SB)], sbufS)

            @pl.when(s > 0)
            def _():
                _wait_scatter(0)
                _wait_scatter(1)
            pltpu.sync_copy(dst_hbm.at[pl.ds(drow, IC), :], dbufS)
            _issue_gather(0, 0)
            _issue_gather(1, 1)
            _issue_aw(awoff, 0, 0)
            _issue_aw(awoff, 1, 1)

            def _chunk(i, par):
                _wait_gather(par)

                @pl.when(i >= 2)
                def _():
                    _wait_scatter(par)
                _compute(i, par)

            def _pair(p, _):
                for par in range(2):
                    i = p * 2 + par
                    _chunk(i, par)
                    _issue_gather(i + 2, par)
                    _issue_aw(awoff, i + 2, par)
                return 0

            lax.fori_loop(0, (IC - 2) // 2, _pair, 0)
            for i in (IC - 2, IC - 1):
                _chunk(i, i % 2)
            return 0
        lax.fori_loop(0, SCN, _super, 0)
        _wait_scatter(0)
        _wait_scatter(1)
        plsc.subcore_barrier()

        pltpu.sync_copy(out_sh.at[pl.ds(rbase, ROWS_T)],
                        out_hbm.at[pl.ds(cid * NN + rbase, ROWS_T)])

    return pl.kernel(
        body,
        out_type=jax.ShapeDtypeStruct((NSC * NN, C), jnp.float32),
        mesh=mesh,
        compiler_params=pltpu.CompilerParams(needs_layout_passes=False),
        scratch_types=[
            pltpu.VMEM((SB,), jnp.int32),
            pltpu.VMEM((IC, B_CH), jnp.int32),
            pltpu.VMEM((B_CH * H,), jnp.float32),
            pltpu.VMEM((B_CH * H,), jnp.float32),
            pltpu.VMEM((B_CH, H * C // 2), jnp.float32),
            pltpu.VMEM((B_CH, H * C // 2), jnp.float32),
            pltpu.VMEM((B_CH, C), jnp.float32),
            pltpu.VMEM((B_CH, C), jnp.float32),
            pltpu.VMEM_SHARED((NN, C), jnp.float32),
            pltpu.SemaphoreType.DMA,
            pltpu.SemaphoreType.DMA,
            pltpu.SemaphoreType.DMA,
            pltpu.SemaphoreType.DMA,
            pltpu.SemaphoreType.DMA,
            pltpu.SemaphoreType.DMA,
        ],
    )(h2, aw_flat, srcp2, dstp2d)


def kernel(x, edge_index, W_m, a_src_m, a_dst_m, b_m,
           W_s, a_src_s, a_dst_s, b_s):
    f32 = jnp.float32

    def fold(W, a):
        return (W.reshape(D, H, C) * a[None, :, :]).sum(-1)

    l = jnp.arange(C, dtype=jnp.int32)
    j2, r = l // 32, l % 32
    colperm = j2 * 32 + (r % 2) * 16 + r // 2
    colperm_all = (jnp.arange(H * C, dtype=jnp.int32) // C) * C + \
        colperm[jnp.arange(H * C) % C]

    w_cat = jnp.concatenate(
        [W_m[:, colperm_all], W_s[:, colperm_all],
         fold(W_m, a_src_m), fold(W_m, a_dst_m),
         fold(W_s, a_src_s), fold(W_s, a_dst_s),
         jnp.zeros((D, W_COLS - 2 * H * C - 4 * H), f32)], axis=1)

    x_pad = jnp.zeros((NN, D), f32).at[:N].set(x)
    hbf, al = _dense_matmul(x_pad, w_cat)
    asrc_flat = jnp.concatenate(
        [al[:, :H].reshape(-1), al[:, 2 * H:3 * H].reshape(-1)])
    adst_flat = jnp.concatenate(
        [al[:, H:2 * H].reshape(-1), al[:, 3 * H:4 * H].reshape(-1)])

    loop = jnp.arange(N, dtype=jnp.int32)
    padv = jnp.full((EP - E - N,), N, jnp.int32)
    srcp = jnp.concatenate([edge_index[0], loop, padv])
    dstp = jnp.concatenate([edge_index[1], loop, padv])
    srcp2 = jnp.concatenate([srcp, srcp + NN])
    dstp2d = dstp.reshape(EP // B_CH, B_CH)

    aw_flat, _, _ = _sc_a(asrc_flat, adst_flat, srcp, dstp)
    h2w = jax.lax.bitcast_convert_type(
        hbf.reshape(NSC * NN, H * C // 2, 2), jnp.float32)
    out2 = _sc_b(h2w, aw_flat, srcp2, dstp2d)

    z_mean = out2[:N] + b_m[None, :]
    z_logstd = out2[NN:NN + N] + b_s[None, :]
    return (z_mean, z_logstd)


# speedup vs baseline: 1.3316x; 1.2792x over previous
import jax
import jax.numpy as jnp
from jax import lax
from jax.experimental import pallas as pl
from jax.experimental.pallas import tpu as pltpu
from jax.experimental.pallas import tpu_sc as plsc

N = 10000
E = 320000
D = 128
H = 4
C = 128

NSC = 2
NTEC = 16
LANES = 16

NN = 10112
NH = NN * H
EP = 331776
EPT = EP // NTEC
A_CH = 768
A_NCH = EPT // A_CH
B_CH = 32
SB = 768
IC = SB // B_CH
SCN = EPT // SB
STRIDE = NH // NTEC
ROWS_T = NN // NTEC
MM_BLK = 1264
W_COLS = 1152


def _bf16_bits(v):
    b = jax.lax.bitcast_convert_type(v, jnp.int32)
    return (b + 0x7FFF + ((b >> 16) & 1)) >> 16


def _mm_body(x_ref, w_ref, hw_ref, al_ref):
    res = jnp.dot(x_ref[...], w_ref[...], preferred_element_type=jnp.float32)
    hc2 = H * C // 2
    for c in range(NSC):
        lo = _bf16_bits(res[:, c * H * C:c * H * C + hc2]) & 0xFFFF
        hi = _bf16_bits(res[:, c * H * C + hc2:(c + 1) * H * C])
        hw_ref[c] = jax.lax.bitcast_convert_type(lo | (hi << 16),
                                                 jnp.float32)
    al_ref[...] = res[:, 2 * H * C:]


def _dense_matmul(x_pad, w_cat):
    return pl.pallas_call(
        _mm_body,
        grid=(NN // MM_BLK,),
        in_specs=[
            pl.BlockSpec((MM_BLK, D), lambda i: (i, 0)),
            pl.BlockSpec((D, W_COLS), lambda i: (0, 0)),
        ],
        out_specs=[
            pl.BlockSpec((NSC, MM_BLK, H * C // 2), lambda i: (0, i, 0)),
            pl.BlockSpec((MM_BLK, W_COLS - 2 * H * C), lambda i: (i, 0)),
        ],
        out_shape=[
            jax.ShapeDtypeStruct((NSC, NN, H * C // 2), jnp.float32),
            jax.ShapeDtypeStruct((NN, W_COLS - 2 * H * C), jnp.float32),
        ],
    )(x_pad, w_cat)


def _lrelu_exp(a):
    return jnp.exp(jnp.where(a >= 0.0, a, 0.2 * a))


def _edge_logits(asrc_t, adst_t, srcb, dstb, j, h):
    sv = srcb[pl.ds(j * LANES, LANES)] * H + h
    dv = dstb[pl.ds(j * LANES, LANES)] * H + h
    a = plsc.load_gather(asrc_t, [sv]) + plsc.load_gather(adst_t, [dv])
    return _lrelu_exp(a), dv


def _sc_a(asrc_flat, adst_flat, srcp, dstp):
    mesh = plsc.VectorSubcoreMesh(core_axis_name="c", subcore_axis_name="s")

    def body(asrc_hbm, adst_hbm, src_hbm, dst_hbm,
             aw_hbm, spart, denom_hbm,
             asrc_t, adst_t, dpart, srcb, dstb, ab, redbuf):
        cid = lax.axis_index("c")
        tid = lax.axis_index("s")
        zero16 = jnp.zeros((LANES,), jnp.float32)

        pltpu.sync_copy(asrc_hbm.at[pl.ds(cid * NH, NH)], asrc_t)
        pltpu.sync_copy(adst_hbm.at[pl.ds(cid * NH, NH)], adst_t)

        def _zero(i, _):
            dpart[pl.ds(i * LANES, LANES)] = zero16
            return 0
        lax.fori_loop(0, NH // LANES, _zero, 0)

        def _p1(ch, _):
            base = tid * EPT + ch * A_CH
            pltpu.sync_copy(src_hbm.at[pl.ds(base, A_CH)], srcb)
            pltpu.sync_copy(dst_hbm.at[pl.ds(base, A_CH)], dstb)

            def _vreg(j, _):
                for h in range(H):
                    e, dv = _edge_logits(asrc_t, adst_t, srcb, dstb, j, h)
                    plsc.addupdate_scatter(dpart, [dv], e)
                return 0
            lax.fori_loop(0, A_CH // LANES, _vreg, 0)
            return 0
        lax.fori_loop(0, A_NCH, _p1, 0)

        cbase = cid * NTEC * NH
        pltpu.sync_copy(dpart, spart.at[pl.ds(cbase + tid * NH, NH)])
        plsc.subcore_barrier()

        def _zr(i, _):
            redbuf[pl.ds(i * LANES, LANES)] = zero16
            return 0
        lax.fori_loop(0, STRIDE // LANES, _zr, 0)
        for p in range(NTEC):
            pltpu.sync_copy(
                spart.at[pl.ds(cbase + p * NH + tid * STRIDE, STRIDE)],
                dpart.at[pl.ds(0, STRIDE)])

            def _acc(i, _):
                redbuf[pl.ds(i * LANES, LANES)] = (
                    redbuf[pl.ds(i * LANES, LANES)]
                    + dpart[pl.ds(i * LANES, LANES)])
                return 0
            lax.fori_loop(0, STRIDE // LANES, _acc, 0)
        pltpu.sync_copy(redbuf,
                        denom_hbm.at[pl.ds(cid * NH + tid * STRIDE, STRIDE)])
        plsc.subcore_barrier()

        pltpu.sync_copy(denom_hbm.at[pl.ds(cid * NH, NH)], dpart)

        iota = lax.broadcasted_iota(jnp.int32, (LANES,), 0)

        def _p2(ch, _):
            base = tid * EPT + ch * A_CH
            pltpu.sync_copy(src_hbm.at[pl.ds(base, A_CH)], srcb)
            pltpu.sync_copy(dst_hbm.at[pl.ds(base, A_CH)], dstb)

            def _vreg(j, _):
                for h in range(H):
                    e, dv = _edge_logits(asrc_t, adst_t, srcb, dstb, j, h)
                    d = plsc.load_gather(dpart, [dv])
                    aw = e / (d + 1e-16) * (1.0 / H)
                    pos = (j * LANES + iota) * H + h
                    plsc.store_scatter(ab, [pos], aw)
                return 0
            lax.fori_loop(0, A_CH // LANES, _vreg, 0)
            pltpu.sync_copy(ab, aw_hbm.at[pl.ds(cid * EP * H + base * H,
                                                A_CH * H)])
            return 0
        lax.fori_loop(0, A_NCH, _p2, 0)

    return pl.kernel(
        body,
        out_type=(
            jax.ShapeDtypeStruct((NSC * EP * H,), jnp.float32),
            jax.ShapeDtypeStruct((NSC * NTEC * NH,), jnp.float32),
            jax.ShapeDtypeStruct((NSC * NH,), jnp.float32),
        ),
        mesh=mesh,
        compiler_params=pltpu.CompilerParams(needs_layout_passes=False),
        scratch_types=[
            pltpu.VMEM((NH,), jnp.float32),
            pltpu.VMEM((NH,), jnp.float32),
            pltpu.VMEM((NH,), jnp.float32),
            pltpu.VMEM((A_CH,), jnp.int32),
            pltpu.VMEM((A_CH,), jnp.int32),
            pltpu.VMEM((A_CH * H,), jnp.float32),
            pltpu.VMEM((STRIDE,), jnp.float32),
        ],
    )(asrc_flat, adst_flat, srcp, dstp)


def _sc_b(h2, aw_flat, srcp2, dstp2d):
    mesh = plsc.VectorSubcoreMesh(core_axis_name="c", subcore_axis_name="s")

    def body(h_hbm, aw_hbm, src_hbm, dst_hbm, out_hbm,
             sbufS, dbufS, awb0, awb1,
             hr0, hr1, ms0, ms1, out_sh,
             gsem0, gsem1, ssem0, ssem1, asem0, asem1):
        cid = lax.axis_index("c")
        tid = lax.axis_index("s")
        zero16 = jnp.zeros((LANES,), jnp.float32)
        hr = (hr0, hr1)
        ms = (ms0, ms1)
        awb = (awb0, awb1)
        gsem = (gsem0, gsem1)
        ssem = (ssem0, ssem1)
        asem = (asem0, asem1)

        def _zm(i, _):
            for k in range(C // LANES):
                ms0[i, pl.ds(k * LANES, LANES)] = zero16
                ms1[i, pl.ds(k * LANES, LANES)] = zero16
            return 0
        lax.fori_loop(0, B_CH, _zm, 0)
        rbase = tid * ROWS_T
        nfull = ROWS_T // B_CH
        for q in range(nfull):
            pltpu.sync_copy(ms0, out_sh.at[pl.ds(rbase + q * B_CH, B_CH)])
        rem = ROWS_T - nfull * B_CH
        if rem:
            pltpu.sync_copy(ms0.at[pl.ds(0, rem)],
                            out_sh.at[pl.ds(rbase + nfull * B_CH, rem)])
        plsc.subcore_barrier()

        def _issue_gather(i, par):
            pltpu.async_copy(
                h_hbm.at[sbufS.at[pl.ds(i * B_CH, B_CH)]], hr[par],
                gsem[par])

        def _wait_gather(par):
            pltpu.make_async_copy(
                h_hbm.at[pl.ds(0, B_CH)], hr[par], gsem[par]).wait()

        def _issue_aw(awoff, i, par):
            pltpu.async_copy(
                aw_hbm.at[pl.ds(awoff + i * (B_CH * H), B_CH * H)],
                awb[par], asem[par])

        def _wait_aw(par):
            pltpu.make_async_copy(
                aw_hbm.at[pl.ds(0, B_CH * H)], awb[par], asem[par]).wait()

        def _wait_scatter(par):
            pltpu.make_async_copy(
                ms[par], out_sh.at[pl.ds(0, B_CH)], ssem[par]).wait()

        def _compute(i, par):
            _wait_aw(par)

            himask = jnp.full((LANES,), -65536, jnp.int32)

            def _edge(q, _):
                b0 = q * 2
                base = jnp.full((LANES,), b0 * H, jnp.int32)
                ws = [plsc.load_gather(awb[par], [base + e * H + h])
                      for e in range(2) for h in range(H)]
                for e in range(2):
                    acc = [zero16] * (C // LANES)
                    for h in range(H):
                        w = ws[e * H + h]
                        for j in range(C // (2 * LANES)):
                            v = plsc.bitcast(
                                hr[par][b0 + e,
                                        pl.ds(h * (C // 2) + j * LANES,
                                              LANES)],
                                jnp.int32)
                            u0 = plsc.bitcast(v << 16, jnp.float32)
                            u1 = plsc.bitcast(v & himask, jnp.float32)
                            acc[2 * j] = acc[2 * j] + w * u0
                            acc[2 * j + 1] = acc[2 * j + 1] + w * u1
                    for k in range(C // LANES):
                        ms[par][b0 + e, pl.ds(k * LANES, LANES)] = acc[k]
                return 0
            lax.fori_loop(0, B_CH // 2, _edge, 0)
            pltpu.async_copy(ms[par], out_sh.at[dbufS.at[i]], ssem[par],
                             add=True)

        def _super(s, _):
            sgbase = tid * EPT + s * SB
            awoff = cid * EP * H + sgbase * H
            drow = pl.multiple_of(tid * (EPT // B_CH) + s * IC, 8)
            pltpu.sync_copy(src_hbm.at[pl.ds(cid * EP + sgbase, SB)], sbufS)

            @pl.when(s > 0)
            def _():
                _wait_scatter(0)
                _wait_scatter(1)
            pltpu.sync_copy(dst_hbm.at[pl.ds(drow, IC), :], dbufS)
            _issue_gather(0, 0)
            _issue_gather(1, 1)
            _issue_aw(awoff, 0, 0)
            _issue_aw(awoff, 1, 1)

            def _chunk(i, par):
                _wait_gather(par)

                @pl.when(i >= 2)
                def _():
                    _wait_scatter(par)
                _compute(i, par)

            def _pair(p, _):
                for par in range(2):
                    i = p * 2 + par
                    _chunk(i, par)
                    _issue_gather(i + 2, par)
                    _issue_aw(awoff, i + 2, par)
                return 0

            lax.fori_loop(0, (IC - 2) // 2, _pair, 0)
            for i in (IC - 2, IC - 1):
                _chunk(i, i % 2)
            return 0
        lax.fori_loop(0, SCN, _super, 0)
        _wait_scatter(0)
        _wait_scatter(1)
        plsc.subcore_barrier()

        pltpu.sync_copy(out_sh.at[pl.ds(rbase, ROWS_T)],
                        out_hbm.at[pl.ds(cid * NN + rbase, ROWS_T)])

    return pl.kernel(
        body,
        out_type=jax.ShapeDtypeStruct((NSC * NN, C), jnp.float32),
        mesh=mesh,
        compiler_params=pltpu.CompilerParams(needs_layout_passes=False),
        scratch_types=[
            pltpu.VMEM((SB,), jnp.int32),
            pltpu.VMEM((IC, B_CH), jnp.int32),
            pltpu.VMEM((B_CH * H,), jnp.float32),
            pltpu.VMEM((B_CH * H,), jnp.float32),
            pltpu.VMEM((B_CH, H * C // 2), jnp.float32),
            pltpu.VMEM((B_CH, H * C // 2), jnp.float32),
            pltpu.VMEM((B_CH, C), jnp.float32),
            pltpu.VMEM((B_CH, C), jnp.float32),
            pltpu.VMEM_SHARED((NN, C), jnp.float32),
            pltpu.SemaphoreType.DMA,
            pltpu.SemaphoreType.DMA,
            pltpu.SemaphoreType.DMA,
            pltpu.SemaphoreType.DMA,
            pltpu.SemaphoreType.DMA,
            pltpu.SemaphoreType.DMA,
        ],
    )(h2, aw_flat, srcp2, dstp2d)


def kernel(x, edge_index, W_m, a_src_m, a_dst_m, b_m,
           W_s, a_src_s, a_dst_s, b_s):
    f32 = jnp.float32

    def fold(W, a):
        return (W.reshape(D, H, C) * a[None, :, :]).sum(-1)

    q = jnp.arange(H * C // 2, dtype=jnp.int32)
    base_ch = (q // 64) * C + ((q % 64) // 16) * 32 + q % 16
    colperm_all = jnp.concatenate([base_ch, base_ch + 16])

    w_cat = jnp.concatenate(
        [W_m[:, colperm_all], W_s[:, colperm_all],
         fold(W_m, a_src_m), fold(W_m, a_dst_m),
         fold(W_s, a_src_s), fold(W_s, a_dst_s),
         jnp.zeros((D, W_COLS - 2 * H * C - 4 * H), f32)], axis=1)

    x_pad = jnp.zeros((NN, D), f32).at[:N].set(x)
    hw, al = _dense_matmul(x_pad, w_cat)
    asrc_flat = jnp.concatenate(
        [al[:, :H].reshape(-1), al[:, 2 * H:3 * H].reshape(-1)])
    adst_flat = jnp.concatenate(
        [al[:, H:2 * H].reshape(-1), al[:, 3 * H:4 * H].reshape(-1)])

    loop = jnp.arange(N, dtype=jnp.int32)
    padv = jnp.full((EP - E - N,), N, jnp.int32)
    srcp = jnp.concatenate([edge_index[0], loop, padv])
    dstp = jnp.concatenate([edge_index[1], loop, padv])
    srcp2 = jnp.concatenate([srcp, srcp + NN])
    dstp2d = dstp.reshape(EP // B_CH, B_CH)

    aw_flat, _, _ = _sc_a(asrc_flat, adst_flat, srcp, dstp)
    out2 = _sc_b(hw.reshape(NSC * NN, H * C // 2), aw_flat, srcp2, dstp2d)

    z_mean = out2[:N] + b_m[None, :]
    z_logstd = out2[NN:NN + N] + b_s[None, :]
    return (z_mean, z_logstd)


# speedup vs baseline: 1.3968x; 1.0489x over previous
import jax
import jax.numpy as jnp
from jax import lax
from jax.experimental import pallas as pl
from jax.experimental.pallas import tpu as pltpu
from jax.experimental.pallas import tpu_sc as plsc

N = 10000
E = 320000
D = 128
H = 4
C = 128

NSC = 2
NTEC = 16
LANES = 16

NN = 10112
NH = NN * H
EP = 331776
EPT = EP // NTEC
A_CH = 576
A_NCH = EPT // A_CH
B_CH = 32
SB = 768
IC = SB // B_CH
SCN = EPT // SB
STRIDE = NH // NTEC
ROWS_T = NN // NTEC
MM_BLK = 1264
W_COLS = 1152


def _bf16_bits(v):
    b = jax.lax.bitcast_convert_type(v, jnp.int32)
    return (b + 0x7FFF + ((b >> 16) & 1)) >> 16


def _mm_body(x_ref, w_ref, hw_ref, al_ref):
    res = jnp.dot(x_ref[...], w_ref[...], preferred_element_type=jnp.float32)
    hc2 = H * C // 2
    for c in range(NSC):
        lo = _bf16_bits(res[:, c * H * C:c * H * C + hc2]) & 0xFFFF
        hi = _bf16_bits(res[:, c * H * C + hc2:(c + 1) * H * C])
        hw_ref[c] = jax.lax.bitcast_convert_type(lo | (hi << 16),
                                                 jnp.float32)
    al_ref[...] = res[:, 2 * H * C:]


def _dense_matmul(x_pad, w_cat):
    return pl.pallas_call(
        _mm_body,
        grid=(NN // MM_BLK,),
        in_specs=[
            pl.BlockSpec((MM_BLK, D), lambda i: (i, 0)),
            pl.BlockSpec((D, W_COLS), lambda i: (0, 0)),
        ],
        out_specs=[
            pl.BlockSpec((NSC, MM_BLK, H * C // 2), lambda i: (0, i, 0)),
            pl.BlockSpec((MM_BLK, W_COLS - 2 * H * C), lambda i: (i, 0)),
        ],
        out_shape=[
            jax.ShapeDtypeStruct((NSC, NN, H * C // 2), jnp.float32),
            jax.ShapeDtypeStruct((NN, W_COLS - 2 * H * C), jnp.float32),
        ],
    )(x_pad, w_cat)


def _lrelu_exp(a):
    return jnp.exp(jnp.where(a >= 0.0, a, 0.2 * a))


def _edge_logits(asrc_t, adst_t, srcb, dstb, j, h):
    sv = srcb[pl.ds(j * LANES, LANES)] * H + h
    dv = dstb[pl.ds(j * LANES, LANES)] * H + h
    a = plsc.load_gather(asrc_t, [sv]) + plsc.load_gather(adst_t, [dv])
    return _lrelu_exp(a), dv


def _sc_a(asrc_flat, adst_flat, srcp, dstp):
    mesh = plsc.VectorSubcoreMesh(core_axis_name="c", subcore_axis_name="s")

    def body(asrc_hbm, adst_hbm, src_hbm, dst_hbm,
             aw_hbm, spart, denom_hbm,
             asrc_t, adst_t, dpart, srcb0, dstb0, srcb1, dstb1,
             ab, redbuf, isem0, isem1):
        cid = lax.axis_index("c")
        tid = lax.axis_index("s")
        zero16 = jnp.zeros((LANES,), jnp.float32)
        srcb_ = (srcb0, srcb1)
        dstb_ = (dstb0, dstb1)
        isem = (isem0, isem1)

        def _issue_idx(ch, par):
            base = tid * EPT + ch * A_CH
            pltpu.async_copy(src_hbm.at[pl.ds(base, A_CH)], srcb_[par],
                             isem[par])
            pltpu.async_copy(dst_hbm.at[pl.ds(base, A_CH)], dstb_[par],
                             isem[par])

        def _wait_idx(par):
            pltpu.make_async_copy(src_hbm.at[pl.ds(0, A_CH)], srcb_[par],
                                  isem[par]).wait()
            pltpu.make_async_copy(dst_hbm.at[pl.ds(0, A_CH)], dstb_[par],
                                  isem[par]).wait()

        def _sweep(chunk_fn):
            _issue_idx(0, 0)

            def _pair(p, _):
                for par in range(2):
                    ch = p * 2 + par
                    _wait_idx(par)

                    @pl.when(ch + 1 < A_NCH)
                    def _():
                        _issue_idx(ch + 1, 1 - par)
                    chunk_fn(ch, srcb_[par], dstb_[par])
                return 0
            lax.fori_loop(0, A_NCH // 2, _pair, 0)

        pltpu.sync_copy(asrc_hbm.at[pl.ds(cid * NH, NH)], asrc_t)
        pltpu.sync_copy(adst_hbm.at[pl.ds(cid * NH, NH)], adst_t)

        def _zero(i, _):
            dpart[pl.ds(i * LANES, LANES)] = zero16
            return 0
        lax.fori_loop(0, NH // LANES, _zero, 0)

        def _p1(ch, srcb, dstb):
            def _vreg(j, _):
                for h in range(H):
                    e, dv = _edge_logits(asrc_t, adst_t, srcb, dstb, j, h)
                    plsc.addupdate_scatter(dpart, [dv], e)
                return 0
            lax.fori_loop(0, A_CH // LANES, _vreg, 0)
        _sweep(_p1)

        cbase = cid * NTEC * NH
        pltpu.sync_copy(dpart, spart.at[pl.ds(cbase + tid * NH, NH)])
        plsc.subcore_barrier()

        def _zr(i, _):
            redbuf[pl.ds(i * LANES, LANES)] = zero16
            return 0
        lax.fori_loop(0, STRIDE // LANES, _zr, 0)
        for p in range(NTEC):
            pltpu.sync_copy(
                spart.at[pl.ds(cbase + p * NH + tid * STRIDE, STRIDE)],
                dpart.at[pl.ds(0, STRIDE)])

            def _acc(i, _):
                redbuf[pl.ds(i * LANES, LANES)] = (
                    redbuf[pl.ds(i * LANES, LANES)]
                    + dpart[pl.ds(i * LANES, LANES)])
                return 0
            lax.fori_loop(0, STRIDE // LANES, _acc, 0)
        pltpu.sync_copy(redbuf,
                        denom_hbm.at[pl.ds(cid * NH + tid * STRIDE, STRIDE)])
        plsc.subcore_barrier()

        pltpu.sync_copy(denom_hbm.at[pl.ds(cid * NH, NH)], dpart)

        iota = lax.broadcasted_iota(jnp.int32, (LANES,), 0)

        def _p2(ch, srcb, dstb):
            def _vreg(j, _):
                for h in range(H):
                    e, dv = _edge_logits(asrc_t, adst_t, srcb, dstb, j, h)
                    d = plsc.load_gather(dpart, [dv])
                    aw = e / (d + 1e-16) * (1.0 / H)
                    pos = (j * LANES + iota) * H + h
                    plsc.store_scatter(ab, [pos], aw)
                return 0
            lax.fori_loop(0, A_CH // LANES, _vreg, 0)
            base = tid * EPT + ch * A_CH
            pltpu.sync_copy(ab, aw_hbm.at[pl.ds(cid * EP * H + base * H,
                                                A_CH * H)])
        _sweep(_p2)

    return pl.kernel(
        body,
        out_type=(
            jax.ShapeDtypeStruct((NSC * EP * H,), jnp.float32),
            jax.ShapeDtypeStruct((NSC * NTEC * NH,), jnp.float32),
            jax.ShapeDtypeStruct((NSC * NH,), jnp.float32),
        ),
        mesh=mesh,
        compiler_params=pltpu.CompilerParams(needs_layout_passes=False),
        scratch_types=[
            pltpu.VMEM((NH,), jnp.float32),
            pltpu.VMEM((NH,), jnp.float32),
            pltpu.VMEM((NH,), jnp.float32),
            pltpu.VMEM((A_CH,), jnp.int32),
            pltpu.VMEM((A_CH,), jnp.int32),
            pltpu.VMEM((A_CH,), jnp.int32),
            pltpu.VMEM((A_CH,), jnp.int32),
            pltpu.VMEM((A_CH * H,), jnp.float32),
            pltpu.VMEM((STRIDE,), jnp.float32),
            pltpu.SemaphoreType.DMA,
            pltpu.SemaphoreType.DMA,
        ],
    )(asrc_flat, adst_flat, srcp, dstp)


def _sc_b(h2, aw_flat, srcp2, dstp2d):
    mesh = plsc.VectorSubcoreMesh(core_axis_name="c", subcore_axis_name="s")

    def body(h_hbm, aw_hbm, src_hbm, dst_hbm, out_hbm,
             sbufS, dbufS, awb0, awb1,
             hr0, hr1, ms0, ms1, out_sh,
             gsem0, gsem1, ssem0, ssem1, asem0, asem1):
        cid = lax.axis_index("c")
        tid = lax.axis_index("s")
        zero16 = jnp.zeros((LANES,), jnp.float32)
        hr = (hr0, hr1)
        ms = (ms0, ms1)
        awb = (awb0, awb1)
        gsem = (gsem0, gsem1)
        ssem = (ssem0, ssem1)
        asem = (asem0, asem1)

        def _zm(i, _):
            for k in range(C // LANES):
                ms0[i, pl.ds(k * LANES, LANES)] = zero16
                ms1[i, pl.ds(k * LANES, LANES)] = zero16
            return 0
        lax.fori_loop(0, B_CH, _zm, 0)
        rbase = tid * ROWS_T
        nfull = ROWS_T // B_CH
        for q in range(nfull):
            pltpu.sync_copy(ms0, out_sh.at[pl.ds(rbase + q * B_CH, B_CH)])
        rem = ROWS_T - nfull * B_CH
        if rem:
            pltpu.sync_copy(ms0.at[pl.ds(0, rem)],
                            out_sh.at[pl.ds(rbase + nfull * B_CH, rem)])
        plsc.subcore_barrier()

        def _issue_gather(i, par):
            pltpu.async_copy(
                h_hbm.at[sbufS.at[pl.ds(i * B_CH, B_CH)]], hr[par],
                gsem[par])

        def _wait_gather(par):
            pltpu.make_async_copy(
                h_hbm.at[pl.ds(0, B_CH)], hr[par], gsem[par]).wait()

        def _issue_aw(awoff, i, par):
            pltpu.async_copy(
                aw_hbm.at[pl.ds(awoff + i * (B_CH * H), B_CH * H)],
                awb[par], asem[par])

        def _wait_aw(par):
            pltpu.make_async_copy(
                aw_hbm.at[pl.ds(0, B_CH * H)], awb[par], asem[par]).wait()

        def _wait_scatter(par):
            pltpu.make_async_copy(
                ms[par], out_sh.at[pl.ds(0, B_CH)], ssem[par]).wait()

        def _compute(i, par):
            _wait_aw(par)

            himask = jnp.full((LANES,), -65536, jnp.int32)

            def _edge(q, _):
                b0 = q * 2
                base = jnp.full((LANES,), b0 * H, jnp.int32)
                ws = [plsc.load_gather(awb[par], [base + e * H + h])
                      for e in range(2) for h in range(H)]
                for e in range(2):
                    acc = [zero16] * (C // LANES)
                    for h in range(H):
                        w = ws[e * H + h]
                        for j in range(C // (2 * LANES)):
                            v = plsc.bitcast(
                                hr[par][b0 + e,
                                        pl.ds(h * (C // 2) + j * LANES,
                                              LANES)],
                                jnp.int32)
                            u0 = plsc.bitcast(v << 16, jnp.float32)
                            u1 = plsc.bitcast(v & himask, jnp.float32)
                            acc[2 * j] = acc[2 * j] + w * u0
                            acc[2 * j + 1] = acc[2 * j + 1] + w * u1
                    for k in range(C // LANES):
                        ms[par][b0 + e, pl.ds(k * LANES, LANES)] = acc[k]
                return 0
            lax.fori_loop(0, B_CH // 2, _edge, 0)
            pltpu.async_copy(ms[par], out_sh.at[dbufS.at[i]], ssem[par],
                             add=True)

        def _super(s, _):
            sgbase = tid * EPT + s * SB
            awoff = cid * EP * H + sgbase * H
            drow = pl.multiple_of(tid * (EPT // B_CH) + s * IC, 8)
            pltpu.sync_copy(src_hbm.at[pl.ds(cid * EP + sgbase, SB)], sbufS)

            @pl.when(s > 0)
            def _():
                _wait_scatter(0)
                _wait_scatter(1)
            pltpu.sync_copy(dst_hbm.at[pl.ds(drow, IC), :], dbufS)
            _issue_gather(0, 0)
            _issue_gather(1, 1)
            _issue_aw(awoff, 0, 0)
            _issue_aw(awoff, 1, 1)

            def _chunk(i, par):
                _wait_gather(par)

                @pl.when(i >= 2)
                def _():
                    _wait_scatter(par)
                _compute(i, par)

            def _pair(p, _):
                for par in range(2):
                    i = p * 2 + par
                    _chunk(i, par)
                    _issue_gather(i + 2, par)
                    _issue_aw(awoff, i + 2, par)
                return 0

            lax.fori_loop(0, (IC - 2) // 2, _pair, 0)
            for i in (IC - 2, IC - 1):
                _chunk(i, i % 2)
            return 0
        lax.fori_loop(0, SCN, _super, 0)
        _wait_scatter(0)
        _wait_scatter(1)
        plsc.subcore_barrier()

        pltpu.sync_copy(out_sh.at[pl.ds(rbase, ROWS_T)],
                        out_hbm.at[pl.ds(cid * NN + rbase, ROWS_T)])

    return pl.kernel(
        body,
        out_type=jax.ShapeDtypeStruct((NSC * NN, C), jnp.float32),
        mesh=mesh,
        compiler_params=pltpu.CompilerParams(needs_layout_passes=False),
        scratch_types=[
            pltpu.VMEM((SB,), jnp.int32),
            pltpu.VMEM((IC, B_CH), jnp.int32),
            pltpu.VMEM((B_CH * H,), jnp.float32),
            pltpu.VMEM((B_CH * H,), jnp.float32),
            pltpu.VMEM((B_CH, H * C // 2), jnp.float32),
            pltpu.VMEM((B_CH, H * C // 2), jnp.float32),
            pltpu.VMEM((B_CH, C), jnp.float32),
            pltpu.VMEM((B_CH, C), jnp.float32),
            pltpu.VMEM_SHARED((NN, C), jnp.float32),
            pltpu.SemaphoreType.DMA,
            pltpu.SemaphoreType.DMA,
            pltpu.SemaphoreType.DMA,
            pltpu.SemaphoreType.DMA,
            pltpu.SemaphoreType.DMA,
            pltpu.SemaphoreType.DMA,
        ],
    )(h2, aw_flat, srcp2, dstp2d)


def kernel(x, edge_index, W_m, a_src_m, a_dst_m, b_m,
           W_s, a_src_s, a_dst_s, b_s):
    f32 = jnp.float32

    def fold(W, a):
        return (W.reshape(D, H, C) * a[None, :, :]).sum(-1)

    q = jnp.arange(H * C // 2, dtype=jnp.int32)
    base_ch = (q // 64) * C + ((q % 64) // 16) * 32 + q % 16
    colperm_all = jnp.concatenate([base_ch, base_ch + 16])

    w_cat = jnp.concatenate(
        [W_m[:, colperm_all], W_s[:, colperm_all],
         fold(W_m, a_src_m), fold(W_m, a_dst_m),
         fold(W_s, a_src_s), fold(W_s, a_dst_s),
         jnp.zeros((D, W_COLS - 2 * H * C - 4 * H), f32)], axis=1)

    x_pad = jnp.zeros((NN, D), f32).at[:N].set(x)
    hw, al = _dense_matmul(x_pad, w_cat)
    asrc_flat = jnp.concatenate(
        [al[:, :H].reshape(-1), al[:, 2 * H:3 * H].reshape(-1)])
    adst_flat = jnp.concatenate(
        [al[:, H:2 * H].reshape(-1), al[:, 3 * H:4 * H].reshape(-1)])

    loop = jnp.arange(N, dtype=jnp.int32)
    padv = jnp.full((EP - E - N,), N, jnp.int32)
    srcp = jnp.concatenate([edge_index[0], loop, padv])
    dstp = jnp.concatenate([edge_index[1], loop, padv])
    srcp2 = jnp.concatenate([srcp, srcp + NN])
    dstp2d = dstp.reshape(EP // B_CH, B_CH)

    aw_flat, _, _ = _sc_a(asrc_flat, adst_flat, srcp, dstp)
    out2 = _sc_b(hw.reshape(NSC * NN, H * C // 2), aw_flat, srcp2, dstp2d)

    z_mean = out2[:N] + b_m[None, :]
    z_logstd = out2[NN:NN + N] + b_s[None, :]
    return (z_mean, z_logstd)


# speedup vs baseline: 1.4235x; 1.0191x over previous
import jax
import jax.numpy as jnp
from jax import lax
from jax.experimental import pallas as pl
from jax.experimental.pallas import tpu as pltpu
from jax.experimental.pallas import tpu_sc as plsc

N = 10000
E = 320000
D = 128
H = 4
C = 128

NSC = 2
NTEC = 16
LANES = 16

NN = 10112
NH = NN * H
EP = 331776
EPT = EP // NTEC
A_CH = 576
A_NCH = EPT // A_CH
B_CH = 32
SB = 768
IC = SB // B_CH
SCN = EPT // SB
STRIDE = NH // NTEC
ROWS_T = NN // NTEC
MM_BLK = 1264
W_COLS = 1152


def _bf16_bits(v):
    b = jax.lax.bitcast_convert_type(v, jnp.int32)
    return (b + 0x7FFF + ((b >> 16) & 1)) >> 16


def _mm_body(x_ref, w_ref, hw_ref, al_ref):
    res = jnp.dot(x_ref[...], w_ref[...], preferred_element_type=jnp.float32)
    hc2 = H * C // 2
    for c in range(NSC):
        lo = _bf16_bits(res[:, c * H * C:c * H * C + hc2]) & 0xFFFF
        hi = _bf16_bits(res[:, c * H * C + hc2:(c + 1) * H * C])
        hw_ref[c] = jax.lax.bitcast_convert_type(lo | (hi << 16),
                                                 jnp.float32)
    al_ref[...] = res[:, 2 * H * C:]


def _dense_matmul(x_pad, w_cat):
    return pl.pallas_call(
        _mm_body,
        grid=(NN // MM_BLK,),
        in_specs=[
            pl.BlockSpec((MM_BLK, D), lambda i: (i, 0)),
            pl.BlockSpec((D, W_COLS), lambda i: (0, 0)),
        ],
        out_specs=[
            pl.BlockSpec((NSC, MM_BLK, H * C // 2), lambda i: (0, i, 0)),
            pl.BlockSpec((MM_BLK, W_COLS - 2 * H * C), lambda i: (i, 0)),
        ],
        out_shape=[
            jax.ShapeDtypeStruct((NSC, NN, H * C // 2), jnp.float32),
            jax.ShapeDtypeStruct((NN, W_COLS - 2 * H * C), jnp.float32),
        ],
    )(x_pad, w_cat)


def _lrelu_exp(a):
    return jnp.exp(jnp.where(a >= 0.0, a, 0.2 * a))


def _edge_logits(asrc_t, adst_t, srcb, dstb, j, h):
    sv = srcb[pl.ds(j * LANES, LANES)] * H + h
    dv = dstb[pl.ds(j * LANES, LANES)] * H + h
    a = plsc.load_gather(asrc_t, [sv]) + plsc.load_gather(adst_t, [dv])
    return _lrelu_exp(a), dv


def _sc_a(asrc_flat, adst_flat, srcp, dstp):
    mesh = plsc.VectorSubcoreMesh(core_axis_name="c", subcore_axis_name="s")

    def body(asrc_hbm, adst_hbm, src_hbm, dst_hbm,
             aw_hbm, spart, denom_hbm,
             asrc_t, adst_t, dpart, srcb0, dstb0, srcb1, dstb1,
             ab, redbuf, isem0, isem1):
        cid = lax.axis_index("c")
        tid = lax.axis_index("s")
        zero16 = jnp.zeros((LANES,), jnp.float32)
        srcb_ = (srcb0, srcb1)
        dstb_ = (dstb0, dstb1)
        isem = (isem0, isem1)

        def _issue_idx(ch, par):
            base = tid * EPT + ch * A_CH
            pltpu.async_copy(src_hbm.at[pl.ds(base, A_CH)], srcb_[par],
                             isem[par])
            pltpu.async_copy(dst_hbm.at[pl.ds(base, A_CH)], dstb_[par],
                             isem[par])

        def _wait_idx(par):
            pltpu.make_async_copy(src_hbm.at[pl.ds(0, A_CH)], srcb_[par],
                                  isem[par]).wait()
            pltpu.make_async_copy(dst_hbm.at[pl.ds(0, A_CH)], dstb_[par],
                                  isem[par]).wait()

        def _sweep(chunk_fn):
            _issue_idx(0, 0)

            def _pair(p, _):
                for par in range(2):
                    ch = p * 2 + par
                    _wait_idx(par)

                    @pl.when(ch + 1 < A_NCH)
                    def _():
                        _issue_idx(ch + 1, 1 - par)
                    chunk_fn(ch, srcb_[par], dstb_[par])
                return 0
            lax.fori_loop(0, A_NCH // 2, _pair, 0)

        pltpu.sync_copy(asrc_hbm.at[pl.ds(cid * NH, NH)], asrc_t)
        pltpu.sync_copy(adst_hbm.at[pl.ds(cid * NH, NH)], adst_t)

        def _zero(i, _):
            dpart[pl.ds(i * LANES, LANES)] = zero16
            return 0
        lax.fori_loop(0, NH // LANES, _zero, 0)

        def _p1(ch, srcb, dstb):
            def _vreg(j, _):
                for h in range(H):
                    e, dv = _edge_logits(asrc_t, adst_t, srcb, dstb, j, h)
                    plsc.addupdate_scatter(dpart, [dv], e)
                return 0
            lax.fori_loop(0, A_CH // LANES, _vreg, 0)
        _sweep(_p1)

        cbase = cid * NTEC * NH
        pltpu.sync_copy(dpart, spart.at[pl.ds(cbase + tid * NH, NH)])
        plsc.subcore_barrier()

        def _zr(i, _):
            redbuf[pl.ds(i * LANES, LANES)] = zero16
            return 0
        lax.fori_loop(0, STRIDE // LANES, _zr, 0)
        for p in range(NTEC):
            pltpu.sync_copy(
                spart.at[pl.ds(cbase + p * NH + tid * STRIDE, STRIDE)],
                dpart.at[pl.ds(0, STRIDE)])

            def _acc(i, _):
                redbuf[pl.ds(i * LANES, LANES)] = (
                    redbuf[pl.ds(i * LANES, LANES)]
                    + dpart[pl.ds(i * LANES, LANES)])
                return 0
            lax.fori_loop(0, STRIDE // LANES, _acc, 0)
        pltpu.sync_copy(redbuf,
                        denom_hbm.at[pl.ds(cid * NH + tid * STRIDE, STRIDE)])
        plsc.subcore_barrier()

        pltpu.sync_copy(denom_hbm.at[pl.ds(cid * NH, NH)], dpart)

        iota = lax.broadcasted_iota(jnp.int32, (LANES,), 0)

        def _p2(ch, srcb, dstb):
            def _vreg(j, _):
                for h in range(H):
                    e, dv = _edge_logits(asrc_t, adst_t, srcb, dstb, j, h)
                    d = plsc.load_gather(dpart, [dv])
                    aw = e / (d + 1e-16) * (1.0 / H)
                    pos = (j * LANES + iota) * H + h
                    plsc.store_scatter(ab, [pos], aw)
                return 0
            lax.fori_loop(0, A_CH // LANES, _vreg, 0)
            base = tid * EPT + ch * A_CH
            pltpu.sync_copy(ab, aw_hbm.at[pl.ds(cid * EP * H + base * H,
                                                A_CH * H)])
        _sweep(_p2)

    return pl.kernel(
        body,
        out_type=(
            jax.ShapeDtypeStruct((NSC * EP * H,), jnp.float32),
            jax.ShapeDtypeStruct((NSC * NTEC * NH,), jnp.float32),
            jax.ShapeDtypeStruct((NSC * NH,), jnp.float32),
        ),
        mesh=mesh,
        compiler_params=pltpu.CompilerParams(needs_layout_passes=False),
        scratch_types=[
            pltpu.VMEM((NH,), jnp.float32),
            pltpu.VMEM((NH,), jnp.float32),
            pltpu.VMEM((NH,), jnp.float32),
            pltpu.VMEM((A_CH,), jnp.int32),
            pltpu.VMEM((A_CH,), jnp.int32),
            pltpu.VMEM((A_CH,), jnp.int32),
            pltpu.VMEM((A_CH,), jnp.int32),
            pltpu.VMEM((A_CH * H,), jnp.float32),
            pltpu.VMEM((STRIDE,), jnp.float32),
            pltpu.SemaphoreType.DMA,
            pltpu.SemaphoreType.DMA,
        ],
    )(asrc_flat, adst_flat, srcp, dstp)


def _sc_b(h2, aw_flat, srcp2, dstp2d):
    mesh = plsc.VectorSubcoreMesh(core_axis_name="c", subcore_axis_name="s")

    def body(h_hbm, aw_hbm, src_hbm, dst_hbm, out_hbm,
             sbufS, dbufS, awb0, awb1,
             hr0, hr1, ms0, ms1, out_sh,
             gsem0, gsem1, ssem0, ssem1, asem0, asem1):
        cid = lax.axis_index("c")
        tid = lax.axis_index("s")
        zero16 = jnp.zeros((LANES,), jnp.float32)
        hr = (hr0, hr1)
        ms = (ms0, ms1)
        awb = (awb0, awb1)
        gsem = (gsem0, gsem1)
        ssem = (ssem0, ssem1)
        asem = (asem0, asem1)

        def _zm(i, _):
            for k in range(C // LANES):
                ms0[i, pl.ds(k * LANES, LANES)] = zero16
                ms1[i, pl.ds(k * LANES, LANES)] = zero16
            return 0
        lax.fori_loop(0, B_CH, _zm, 0)
        rbase = tid * ROWS_T
        nfull = ROWS_T // B_CH
        for q in range(nfull):
            pltpu.sync_copy(ms0, out_sh.at[pl.ds(rbase + q * B_CH, B_CH)])
        rem = ROWS_T - nfull * B_CH
        if rem:
            pltpu.sync_copy(ms0.at[pl.ds(0, rem)],
                            out_sh.at[pl.ds(rbase + nfull * B_CH, rem)])
        plsc.subcore_barrier()

        def _issue_gather(i, par):
            pltpu.async_copy(
                h_hbm.at[sbufS.at[pl.ds(i * B_CH, B_CH)]], hr[par],
                gsem[par])

        def _wait_gather(par):
            pltpu.make_async_copy(
                h_hbm.at[pl.ds(0, B_CH)], hr[par], gsem[par]).wait()

        def _issue_aw(awoff, i, par):
            pltpu.async_copy(
                aw_hbm.at[pl.ds(awoff + i * (B_CH * H), B_CH * H)],
                awb[par], asem[par])

        def _wait_aw(par):
            pltpu.make_async_copy(
                aw_hbm.at[pl.ds(0, B_CH * H)], awb[par], asem[par]).wait()

        def _wait_scatter(par):
            pltpu.make_async_copy(
                ms[par], out_sh.at[pl.ds(0, B_CH)], ssem[par]).wait()

        def _compute(i, par):
            _wait_aw(par)

            himask = jnp.full((LANES,), -65536, jnp.int32)

            def _edge(q, _):
                b0 = q * 4
                base = jnp.full((LANES,), b0 * H, jnp.int32)
                ws = [plsc.load_gather(awb[par], [base + e * H + h])
                      for e in range(4) for h in range(H)]
                for e in range(4):
                    acc = [zero16] * (C // LANES)
                    for h in range(H):
                        w = ws[e * H + h]
                        for j in range(C // (2 * LANES)):
                            v = plsc.bitcast(
                                hr[par][b0 + e,
                                        pl.ds(h * (C // 2) + j * LANES,
                                              LANES)],
                                jnp.int32)
                            u0 = plsc.bitcast(v << 16, jnp.float32)
                            u1 = plsc.bitcast(v & himask, jnp.float32)
                            acc[2 * j] = acc[2 * j] + w * u0
                            acc[2 * j + 1] = acc[2 * j + 1] + w * u1
                    for k in range(C // LANES):
                        ms[par][b0 + e, pl.ds(k * LANES, LANES)] = acc[k]
                return 0
            lax.fori_loop(0, B_CH // 4, _edge, 0)
            pltpu.async_copy(ms[par], out_sh.at[dbufS.at[i]], ssem[par],
                             add=True)

        def _super(s, _):
            sgbase = tid * EPT + s * SB
            awoff = cid * EP * H + sgbase * H
            drow = pl.multiple_of(tid * (EPT // B_CH) + s * IC, 8)
            pltpu.sync_copy(src_hbm.at[pl.ds(cid * EP + sgbase, SB)], sbufS)

            @pl.when(s > 0)
            def _():
                _wait_scatter(0)
                _wait_scatter(1)
            pltpu.sync_copy(dst_hbm.at[pl.ds(drow, IC), :], dbufS)
            _issue_gather(0, 0)
            _issue_gather(1, 1)
            _issue_aw(awoff, 0, 0)
            _issue_aw(awoff, 1, 1)

            def _chunk(i, par):
                _wait_gather(par)

                @pl.when(i >= 2)
                def _():
                    _wait_scatter(par)
                _compute(i, par)

            def _pair(p, _):
                for par in range(2):
                    i = p * 2 + par
                    _chunk(i, par)
                    _issue_gather(i + 2, par)
                    _issue_aw(awoff, i + 2, par)
                return 0

            lax.fori_loop(0, (IC - 2) // 2, _pair, 0)
            for i in (IC - 2, IC - 1):
                _chunk(i, i % 2)
            return 0
        lax.fori_loop(0, SCN, _super, 0)
        _wait_scatter(0)
        _wait_scatter(1)
        plsc.subcore_barrier()

        pltpu.sync_copy(out_sh.at[pl.ds(rbase, ROWS_T)],
                        out_hbm.at[pl.ds(cid * NN + rbase, ROWS_T)])

    return pl.kernel(
        body,
        out_type=jax.ShapeDtypeStruct((NSC * NN, C), jnp.float32),
        mesh=mesh,
        compiler_params=pltpu.CompilerParams(needs_layout_passes=False),
        scratch_types=[
            pltpu.VMEM((SB,), jnp.int32),
            pltpu.VMEM((IC, B_CH), jnp.int32),
            pltpu.VMEM((B_CH * H,), jnp.float32),
            pltpu.VMEM((B_CH * H,), jnp.float32),
            pltpu.VMEM((B_CH, H * C // 2), jnp.float32),
            pltpu.VMEM((B_CH, H * C // 2), jnp.float32),
            pltpu.VMEM((B_CH, C), jnp.float32),
            pltpu.VMEM((B_CH, C), jnp.float32),
            pltpu.VMEM_SHARED((NN, C), jnp.float32),
            pltpu.SemaphoreType.DMA,
            pltpu.SemaphoreType.DMA,
            pltpu.SemaphoreType.DMA,
            pltpu.SemaphoreType.DMA,
            pltpu.SemaphoreType.DMA,
            pltpu.SemaphoreType.DMA,
        ],
    )(h2, aw_flat, srcp2, dstp2d)


def kernel(x, edge_index, W_m, a_src_m, a_dst_m, b_m,
           W_s, a_src_s, a_dst_s, b_s):
    f32 = jnp.float32

    def fold(W, a):
        return (W.reshape(D, H, C) * a[None, :, :]).sum(-1)

    q = jnp.arange(H * C // 2, dtype=jnp.int32)
    base_ch = (q // 64) * C + ((q % 64) // 16) * 32 + q % 16
    colperm_all = jnp.concatenate([base_ch, base_ch + 16])

    w_cat = jnp.concatenate(
        [W_m[:, colperm_all], W_s[:, colperm_all],
         fold(W_m, a_src_m), fold(W_m, a_dst_m),
         fold(W_s, a_src_s), fold(W_s, a_dst_s),
         jnp.zeros((D, W_COLS - 2 * H * C - 4 * H), f32)], axis=1)

    x_pad = jnp.zeros((NN, D), f32).at[:N].set(x)
    hw, al = _dense_matmul(x_pad, w_cat)
    asrc_flat = jnp.concatenate(
        [al[:, :H].reshape(-1), al[:, 2 * H:3 * H].reshape(-1)])
    adst_flat = jnp.concatenate(
        [al[:, H:2 * H].reshape(-1), al[:, 3 * H:4 * H].reshape(-1)])

    loop = jnp.arange(N, dtype=jnp.int32)
    padv = jnp.full((EP - E - N,), N, jnp.int32)
    srcp = jnp.concatenate([edge_index[0], loop, padv])
    dstp = jnp.concatenate([edge_index[1], loop, padv])
    srcp2 = jnp.concatenate([srcp, srcp + NN])
    dstp2d = dstp.reshape(EP // B_CH, B_CH)

    aw_flat, _, _ = _sc_a(asrc_flat, adst_flat, srcp, dstp)
    out2 = _sc_b(hw.reshape(NSC * NN, H * C // 2), aw_flat, srcp2, dstp2d)

    z_mean = out2[:N] + b_m[None, :]
    z_logstd = out2[NN:NN + N] + b_s[None, :]
    return (z_mean, z_logstd)
